# baseline (speedup 1.0000x reference)
import functools

import jax
import jax.numpy as jnp
from jax import lax
from jax.experimental import pallas as pl
from jax.experimental.pallas import tpu as pltpu
from jax.experimental.pallas import tpu_sc as plsc

N_VOCAB = 49408
D = 768
T = 77
BATCH = 1024
B = BATCH * T

NC = 2
NS = 16
NW = NC * NS
BPW = B // NW
CH = 32
NCH = BPW // CH
LANES = 16
KREG = D // LANES


def _body(tok_hbm, table_hbm, pos_hbm, out_hbm, idx_v, pos_v, rows_v,
          sem_i, sem_p, sem_g, sem_o):
    cid = lax.axis_index("c")
    sid = lax.axis_index("s")
    wid = sid * NC + cid
    base = wid * BPW

    cp_i = pltpu.make_async_copy(tok_hbm.at[pl.ds(base, BPW)], idx_v, sem_i)
    cp_i.start()
    cp_p = pltpu.make_async_copy(pos_hbm, pos_v, sem_p)
    cp_p.start()
    cp_i.wait()
    cp_p.wait()

    t0 = lax.rem(base, T)

    def chunk_body(j, t_carry):
        cp_g = pltpu.make_async_copy(
            table_hbm.at[idx_v.at[pl.ds(j * CH, CH)]], rows_v, sem_g)
        cp_g.start()
        cp_g.wait()

        def row_body(r, t):
            for k in range(KREG):
                sl = pl.ds(k * LANES, LANES)
                plsc.addupdate(rows_v.at[r, sl], pos_v[t, sl])
            t = t + 1
            return jnp.where(t == T, 0, t)

        t_next = lax.fori_loop(0, CH, row_body, t_carry, unroll=False)

        cp_o = pltpu.make_async_copy(
            rows_v, out_hbm.at[pl.ds(base + j * CH, CH)], sem_o)
        cp_o.start()
        cp_o.wait()
        return t_next

    lax.fori_loop(0, NCH, chunk_body, t0, unroll=False)


@jax.jit
def _emb(tok, table, pos):
    kfn = pl.kernel(
        _body,
        out_type=jax.ShapeDtypeStruct((B, D), jnp.float32),
        mesh=plsc.VectorSubcoreMesh(core_axis_name="c", subcore_axis_name="s"),
        scratch_types=[
            pltpu.VMEM((BPW,), jnp.int32),
            pltpu.VMEM((T, D), jnp.float32),
            pltpu.VMEM((CH, D), jnp.float32),
            pltpu.SemaphoreType.DMA,
            pltpu.SemaphoreType.DMA,
            pltpu.SemaphoreType.DMA,
            pltpu.SemaphoreType.DMA,
        ],
    )
    return kfn(tok, table, pos)


def kernel(token, token_embedding, pos_embedding):
    tok = token.reshape(-1).astype(jnp.int32)
    out = _emb(tok, token_embedding, pos_embedding)
    return out.reshape(BATCH, T, D)


# speedup vs baseline: 2.8930x; 2.8930x over previous
import jax
import jax.numpy as jnp
from jax import lax
from jax.experimental import pallas as pl
from jax.experimental.pallas import tpu as pltpu
from jax.experimental.pallas import tpu_sc as plsc

N_VOCAB = 49408
D = 768
T = 77
BATCH = 1024
B = BATCH * T

NC = 2
NS = 16
NW = NC * NS
BPW = B // NW
CH = 32
NCH = BPW // CH
LANES = 16
KREG = D // LANES
KB = 8
NBUF = 4
AHEAD = 2


def _body(tokT_hbm, table_hbm, posw_hbm, out_hbm, idx_v, pos_v, rows_v,
          sem_i, sem_p, sem_g, sem_o):
    cid = lax.axis_index("c")
    sid = lax.axis_index("s")
    wid = sid * NC + cid
    base = wid * BPW
    tlo = base // BATCH

    cp_i = pltpu.make_async_copy(tokT_hbm.at[pl.ds(base, BPW)], idx_v, sem_i)
    cp_i.start()
    poff = pl.multiple_of(wid * 8, 8)
    cp_p = pltpu.make_async_copy(posw_hbm.at[pl.ds(poff, 8)], pos_v, sem_p)
    cp_p.start()
    cp_i.wait()
    cp_p.wait()

    def gather(j, slot):
        pltpu.make_async_copy(
            table_hbm.at[idx_v.at[pl.ds(j * CH, CH)]],
            rows_v.at[slot], sem_g.at[slot]).start()

    for j in range(AHEAD):
        gather(j, j % NBUF)

    def out_copy(slot, b0, t):
        return pltpu.make_async_copy(
            rows_v.at[slot], out_hbm.at[pl.ds(b0, CH), t], sem_o.at[slot])

    def chunk_body(j, _):
        slot = lax.rem(j, NBUF)
        g = base + j * CH
        t = g // BATCH
        b0 = lax.rem(g, BATCH)

        pltpu.make_async_copy(
            table_hbm.at[idx_v.at[pl.ds(j * CH, CH)]],
            rows_v.at[slot], sem_g.at[slot]).wait()

        tl = t - tlo
        for kb in range(KREG // KB):
            pv = [pos_v[tl, pl.ds((kb * KB + k) * LANES, LANES)]
                  for k in range(KB)]

            def row_body(r, c):
                for k in range(KB):
                    plsc.addupdate(
                        rows_v.at[slot, r, pl.ds((kb * KB + k) * LANES, LANES)],
                        pv[k])
                return c

            lax.fori_loop(0, CH, row_body, 0, unroll=2)

        out_copy(slot, b0, t).start()

        nj = j + AHEAD

        @pl.when(nj < NCH)
        def _():
            nslot = lax.rem(nj, NBUF)

            @pl.when(nj >= NBUF)
            def _():
                out_copy(nslot, 0, 0).wait()

            gather(nj, nslot)

        return 0

    lax.fori_loop(0, NCH, chunk_body, 0, unroll=False)

    for b in range(NBUF):
        out_copy(jnp.int32(b), 0, 0).wait()


def _emb(tokT, table, posw):
    kfn = pl.kernel(
        _body,
        out_type=jax.ShapeDtypeStruct((BATCH, T, D), jnp.float32),
        mesh=plsc.VectorSubcoreMesh(core_axis_name="c", subcore_axis_name="s"),
        scratch_types=[
            pltpu.VMEM((BPW,), jnp.int32),
            pltpu.VMEM((8, D), jnp.float32),
            pltpu.VMEM((NBUF, CH, D), jnp.float32),
            pltpu.SemaphoreType.DMA,
            pltpu.SemaphoreType.DMA,
            pltpu.SemaphoreType.DMA((NBUF,)),
            pltpu.SemaphoreType.DMA((NBUF,)),
        ],
    )
    return kfn(tokT, table, posw)


def kernel(token, token_embedding, pos_embedding):
    tokT = token.T.reshape(-1).astype(jnp.int32)
    tlo = (jnp.arange(NW, dtype=jnp.int32) * BPW) // BATCH
    ridx = jnp.minimum(tlo[:, None] + jnp.arange(8, dtype=jnp.int32)[None, :],
                       T - 1).reshape(-1)
    posw = pos_embedding[ridx]
    return _emb(tokT, token_embedding, posw)


# speedup vs baseline: 5.2833x; 1.8262x over previous
import jax
import jax.numpy as jnp
from jax import lax
from jax.experimental import pallas as pl
from jax.experimental.pallas import tpu as pltpu
from jax.experimental.pallas import tpu_sc as plsc

N_VOCAB = 49408
D = 768
T = 77
BATCH = 1024
B = BATCH * T

NC = 2
NS = 16
NW = NC * NS
BPW = B // NW
CH = 32
NCH = BPW // CH
LANES = 16
KREG = D // LANES
KB = 8
NBUF = 4
AHEAD = 2


def _body(tokT_hbm, table_hbm, posw_hbm, out_hbm, idx_v, pos_v, rows_v,
          sem_i, sem_p, sem_g, sem_o):
    cid = lax.axis_index("c")
    sid = lax.axis_index("s")
    wid = sid * NC + cid
    base = wid * BPW
    tlo = base // BATCH

    cp_i = pltpu.make_async_copy(tokT_hbm.at[pl.ds(base, BPW)], idx_v, sem_i)
    cp_i.start()
    poff = pl.multiple_of(wid * 8, 8)
    cp_p = pltpu.make_async_copy(posw_hbm.at[pl.ds(poff, 8)], pos_v, sem_p)
    cp_p.start()
    cp_i.wait()
    cp_p.wait()

    def gather(j, slot):
        pltpu.make_async_copy(
            table_hbm.at[idx_v.at[pl.ds(j * CH, CH)]],
            rows_v.at[slot], sem_g.at[slot]).start()

    for j in range(AHEAD):
        gather(j, j % NBUF)

    def out_copy(slot, b0, t):
        return pltpu.make_async_copy(
            rows_v.at[slot], out_hbm.at[t, pl.ds(b0, CH)], sem_o.at[slot])

    def chunk_body(j, _):
        slot = lax.rem(j, NBUF)
        g = base + j * CH
        t = g // BATCH
        b0 = lax.rem(g, BATCH)

        pltpu.make_async_copy(
            table_hbm.at[idx_v.at[pl.ds(j * CH, CH)]],
            rows_v.at[slot], sem_g.at[slot]).wait()

        tl = t - tlo
        for kb in range(KREG // KB):
            pv = [pos_v[tl, pl.ds((kb * KB + k) * LANES, LANES)]
                  for k in range(KB)]

            def row_body(r, c):
                for k in range(KB):
                    plsc.addupdate(
                        rows_v.at[slot, r, pl.ds((kb * KB + k) * LANES, LANES)],
                        pv[k])
                return c

            lax.fori_loop(0, CH, row_body, 0, unroll=2)

        out_copy(slot, b0, t).start()

        nj = j + AHEAD

        @pl.when(nj < NCH)
        def _():
            nslot = lax.rem(nj, NBUF)

            @pl.when(nj >= NBUF)
            def _():
                out_copy(nslot, 0, 0).wait()

            gather(nj, nslot)

        return 0

    lax.fori_loop(0, NCH, chunk_body, 0, unroll=False)

    for b in range(NBUF):
        out_copy(jnp.int32(b), 0, 0).wait()


def _emb(tokT, table, posw):
    kfn = pl.kernel(
        _body,
        out_type=jax.ShapeDtypeStruct((T, BATCH, D), jnp.float32),
        mesh=plsc.VectorSubcoreMesh(core_axis_name="c", subcore_axis_name="s"),
        scratch_types=[
            pltpu.VMEM((BPW,), jnp.int32),
            pltpu.VMEM((8, D), jnp.float32),
            pltpu.VMEM((NBUF, CH, D), jnp.float32),
            pltpu.SemaphoreType.DMA,
            pltpu.SemaphoreType.DMA,
            pltpu.SemaphoreType.DMA((NBUF,)),
            pltpu.SemaphoreType.DMA((NBUF,)),
        ],
    )
    return kfn(tokT, table, posw)


def kernel(token, token_embedding, pos_embedding):
    tokT = token.T.reshape(-1).astype(jnp.int32)
    tlo = (jnp.arange(NW, dtype=jnp.int32) * BPW) // BATCH
    ridx = jnp.minimum(tlo[:, None] + jnp.arange(8, dtype=jnp.int32)[None, :],
                       T - 1).reshape(-1)
    posw = pos_embedding[ridx]
    return jnp.transpose(_emb(tokT, token_embedding, posw), (1, 0, 2))
